# baseline (speedup 1.0000x reference)
import functools
import math

import jax
import jax.numpy as jnp
from jax.experimental import pallas as pl
from jax.experimental.pallas import tpu as pltpu

HID = 2048
NH = 16
NKV = 4
G = NH // NKV
DQK = 128
DV = 128
S = 2048
BLOCK = 32
TOPN = 8
WIN = 512
T = S // BLOCK
OUT_DIM = 3120
N_ROPE_TILES = (NH + NKV)

_SCALE = 1.0 / math.sqrt(DQK)

QB = 512
KB = 512
RG = QB * G
NQ = S // QB
NKJ = S // KB
BPC = KB // BLOCK
WCH = WIN // KB


def _proj_rope_body(x_ref, w_ref, cos_ref, sin_ref, o_ref):
    j = pl.program_id(1)
    acc = jax.lax.dot_general(
        x_ref[...], w_ref[...], (((1,), (1,)), ((), ())),
        preferred_element_type=jnp.float32)
    c = cos_ref[...]
    s = sin_ref[...]
    a1 = acc[:, : DQK // 2]
    a2 = acc[:, DQK // 2:]
    rot = jnp.concatenate([-a2, a1], axis=1)
    roped = acc * c + rot * s
    o_ref[...] = jnp.where(j < N_ROPE_TILES, roped, acc)


def _proj_rope(x, w_qkvw, cos, sin):
    TS = 2048
    grid = (S // TS, pl.cdiv(OUT_DIM, DQK))
    return pl.pallas_call(
        _proj_rope_body,
        grid=grid,
        in_specs=[
            pl.BlockSpec((TS, HID), lambda i, j: (i, 0)),
            pl.BlockSpec((DQK, HID), lambda i, j: (j, 0)),
            pl.BlockSpec((TS, DQK), lambda i, j: (i, 0)),
            pl.BlockSpec((TS, DQK), lambda i, j: (i, 0)),
        ],
        out_specs=pl.BlockSpec((TS, DQK), lambda i, j: (i, j)),
        out_shape=jax.ShapeDtypeStruct((S, OUT_DIM), jnp.float32),
    )(x, w_qkvw, cos, sin)


def _pool_body(k_ref, v_ref, kc_ref, vc_ref):
    kc_ref[0] = k_ref[0].reshape(T, BLOCK, DQK).mean(axis=1)
    vc_ref[0] = v_ref[0].reshape(T, BLOCK, DV).mean(axis=1)


def _pool(k, v):
    return pl.pallas_call(
        _pool_body,
        grid=(NKV,),
        in_specs=[
            pl.BlockSpec((1, S, DQK), lambda h: (h, 0, 0)),
            pl.BlockSpec((1, S, DV), lambda h: (h, 0, 0)),
        ],
        out_specs=[
            pl.BlockSpec((1, T, DQK), lambda h: (h, 0, 0)),
            pl.BlockSpec((1, T, DV), lambda h: (h, 0, 0)),
        ],
        out_shape=[
            jax.ShapeDtypeStruct((NKV, T, DQK), jnp.float32),
            jax.ShapeDtypeStruct((NKV, T, DV), jnp.float32),
        ],
    )(k, v)


def _attn_body(q_ref, k_ref, v_ref, kc_ref, vc_ref, w_ref, o_ref,
               ocmp_ref, sel_ref, qh_ref,
               msel_ref, lsel_ref, asel_ref,
               mwin_ref, lwin_ref, awin_ref):
    i = pl.program_id(1)
    j = pl.program_id(2)
    q0 = i * QB

    @pl.when(j == 0)
    def _first():
        qg = q_ref[...].reshape(RG, DQK)
        kc = kc_ref[0]
        vc = vc_ref[0]
        s_cmp = jax.lax.dot_general(
            qg, kc, (((1,), (1,)), ((), ())),
            preferred_element_type=jnp.float32) * _SCALE
        iic = q0 + jax.lax.broadcasted_iota(jnp.int32, (RG, T), 0) // G
        jbc = jax.lax.broadcasted_iota(jnp.int32, (RG, T), 1)
        s_cmp = jnp.where(jbc * BLOCK <= iic, s_cmp, -1e9)
        m = jnp.max(s_cmp, axis=1, keepdims=True)
        p = jnp.exp(s_cmp - m)
        p = p / jnp.sum(p, axis=1, keepdims=True)
        ocmp_ref[...] = jnp.dot(p, vc, preferred_element_type=jnp.float32)

        imp = p.reshape(QB, G, T).sum(axis=1)
        iota_f = jax.lax.broadcasted_iota(
            jnp.int32, (QB, T), 1).astype(jnp.float32)
        selk = jnp.zeros((QB, T), jnp.float32)
        work = imp
        for _ in range(TOPN):
            mx = jnp.max(work, axis=1, keepdims=True)
            cand = jnp.where(work == mx, iota_f, jnp.float32(T))
            js = jnp.min(cand, axis=1, keepdims=True)
            hit = iota_f == js
            selk = jnp.where(hit, 1.0, selk)
            work = jnp.where(hit, -jnp.inf, work)
        selg = jnp.broadcast_to(selk[:, None, :], (QB, G, T)).reshape(RG, T)
        sel_ref[...] = selg

        qh_ref[...] = qg * _SCALE
        msel_ref[...] = jnp.full((RG, 1), -1e9, jnp.float32)
        lsel_ref[...] = jnp.zeros((RG, 1), jnp.float32)
        asel_ref[...] = jnp.zeros((RG, DV), jnp.float32)
        mwin_ref[...] = jnp.full((RG, 1), -1e9, jnp.float32)
        lwin_ref[...] = jnp.zeros((RG, 1), jnp.float32)
        awin_ref[...] = jnp.zeros((RG, DV), jnp.float32)

    @pl.when(j <= i)
    def _chunk():
        qh = qh_ref[...]
        kh = k_ref[0]
        vh = v_ref[0]
        ones8 = jnp.ones((KB, 8), jnp.float32)
        s = jax.lax.dot_general(
            qh, kh, (((1,), (1,)), ((), ())),
            preferred_element_type=jnp.float32)

        def _mask_diag(sv):
            cc = j * KB + jax.lax.broadcasted_iota(jnp.int32, (RG, KB), 1)
            rr = q0 + jax.lax.broadcasted_iota(jnp.int32, (RG, KB), 0) // G
            return jnp.where(cc <= rr, sv, jnp.float32(-1e9))

        s = jax.lax.cond(j == i, _mask_diag, lambda sv: sv, s)

        mc = jnp.max(s, axis=1, keepdims=True)
        e16 = jnp.exp(s - mc)

        tt = jax.lax.broadcasted_iota(jnp.int32, (T, KB), 0)
        kb = j * BPC + jax.lax.broadcasted_iota(jnp.int32, (T, KB), 1) // BLOCK
        onehot = (tt == kb).astype(jnp.float32)
        selm = jax.lax.dot_general(
            sel_ref[...], onehot, (((1,), (0,)), ((), ())),
            preferred_element_type=jnp.float32)

        def _update(p16, m_ref, l_ref, a_ref):
            m_old = m_ref[...]
            m_new = jnp.maximum(m_old, mc)
            alpha = jnp.exp(m_old - m_new)
            beta = jnp.exp(mc - m_new)
            rsum = jax.lax.dot_general(
                p16, ones8, (((1,), (0,)), ((), ())),
                preferred_element_type=jnp.float32)[:, 0:1]
            pv = jax.lax.dot_general(
                p16, vh, (((1,), (0,)), ((), ())),
                preferred_element_type=jnp.float32)
            l_ref[...] = l_ref[...] * alpha + rsum * beta
            a_ref[...] = a_ref[...] * alpha + pv * beta
            m_ref[...] = m_new

        _update(e16 * selm, msel_ref, lsel_ref, asel_ref)

        @pl.when(j + WCH >= i)
        def _win():
            def _mask_band(ev):
                rr = q0 + jax.lax.broadcasted_iota(jnp.int32, (RG, KB), 0) // G
                cc = j * KB + jax.lax.broadcasted_iota(jnp.int32, (RG, KB), 1)
                wm = jnp.where(rr - cc < WIN, 1.0, 0.0).astype(jnp.float32)
                return ev * wm

            pw = jax.lax.cond(j + WCH == i, _mask_band, lambda ev: ev, e16)
            _update(pw, mwin_ref, lwin_ref, awin_ref)

        @pl.when(j == i)
        def _fin():
            g = jax.nn.sigmoid(w_ref[...].reshape(RG, 3))
            o_sel = asel_ref[...] * (g[:, 1:2] / lsel_ref[...])
            o_win = awin_ref[...] * (g[:, 2:3] / lwin_ref[...])
            out = g[:, 0:1] * ocmp_ref[...] + o_sel + o_win
            o_ref[...] = out.reshape(QB, 1, G, DV)


def _attn(q, k, v, kc, vc, w):
    grid = (NKV, NQ, NKJ)
    return pl.pallas_call(
        _attn_body,
        grid=grid,
        in_specs=[
            pl.BlockSpec((QB, 1, G, DQK), lambda h, i, j: (i, h, 0, 0)),
            pl.BlockSpec((1, KB, DQK),
                         lambda h, i, j: (h, jnp.minimum(j, i), 0)),
            pl.BlockSpec((1, KB, DV),
                         lambda h, i, j: (h, jnp.minimum(j, i), 0)),
            pl.BlockSpec((1, T, DQK), lambda h, i, j: (h, 0, 0)),
            pl.BlockSpec((1, T, DV), lambda h, i, j: (h, 0, 0)),
            pl.BlockSpec((QB, 1, G, 3), lambda h, i, j: (i, h, 0, 0)),
        ],
        out_specs=pl.BlockSpec((QB, 1, G, DV), lambda h, i, j: (i, h, 0, 0)),
        out_shape=jax.ShapeDtypeStruct((S, NKV, G, DV), jnp.float32),
        scratch_shapes=[
            pltpu.VMEM((RG, DV), jnp.float32),
            pltpu.VMEM((RG, T), jnp.float32),
            pltpu.VMEM((RG, DQK), jnp.float32),
            pltpu.VMEM((RG, 1), jnp.float32),
            pltpu.VMEM((RG, 1), jnp.float32),
            pltpu.VMEM((RG, DV), jnp.float32),
            pltpu.VMEM((RG, 1), jnp.float32),
            pltpu.VMEM((RG, 1), jnp.float32),
            pltpu.VMEM((RG, DV), jnp.float32),
        ],
    )(q, k, v, kc, vc, w)


def _out_proj_body(x_ref, w_ref, o_ref):
    o_ref[...] = jax.lax.dot_general(
        x_ref[...], w_ref[...],
        (((1,), (1,)), ((), ())),
        preferred_element_type=jnp.float32)


def _out_proj(x, w_o):
    TS = 2048
    TN = 512
    grid = (S // TS, HID // TN)
    return pl.pallas_call(
        _out_proj_body,
        grid=grid,
        in_specs=[
            pl.BlockSpec((TS, NH * DV), lambda i, j: (i, 0)),
            pl.BlockSpec((TN, NH * DV), lambda i, j: (j, 0)),
        ],
        out_specs=pl.BlockSpec((TS, TN), lambda i, j: (i, j)),
        out_shape=jax.ShapeDtypeStruct((S, HID), jnp.float32),
    )(x, w_o)


def kernel(hidden_states, cos, sin, cu_seqlens, W_qkvw, W_o):
    s, b, d = hidden_states.shape
    x = hidden_states.reshape(s, d)
    qkvw = _proj_rope(x, W_qkvw, cos, sin)
    o0 = NH * DQK
    o1 = o0 + NKV * DQK
    o2 = o1 + NKV * DV
    q = qkvw[:, :o0].reshape(S, NKV, G, DQK)
    k = qkvw[:, o0:o1].reshape(S, NKV, DQK).transpose(1, 0, 2)
    v = qkvw[:, o1:o2].reshape(S, NKV, DV).transpose(1, 0, 2)
    w = qkvw[:, o2:].reshape(S, NKV, G, 3)
    kc, vc = _pool(k, v)
    o = _attn(q, k, v, kc, vc, w)
    out = _out_proj(o.reshape(S, NH * DV), W_o)
    return out.reshape(s, b, HID)


# speedup vs baseline: 1.0657x; 1.0657x over previous
import functools
import math

import jax
import jax.numpy as jnp
from jax.experimental import pallas as pl
from jax.experimental.pallas import tpu as pltpu

HID = 2048
NH = 16
NKV = 4
G = NH // NKV
DQK = 128
DV = 128
S = 2048
BLOCK = 32
TOPN = 8
WIN = 512
T = S // BLOCK
OUT_DIM = 3120
N_ROPE_TILES = (NH + NKV)

_SCALE = 1.0 / math.sqrt(DQK)

QB = 512
KB = 512
RG = QB * G
NQ = S // QB
NKJ = S // KB
BPC = KB // BLOCK
WCH = WIN // KB


def _proj_rope_body(x_ref, w_ref, cos_ref, sin_ref, o_ref):
    j = pl.program_id(1)
    acc = jax.lax.dot_general(
        x_ref[...], w_ref[...], (((1,), (1,)), ((), ())),
        preferred_element_type=jnp.float32)
    c = cos_ref[...]
    s = sin_ref[...]
    a1 = acc[:, : DQK // 2]
    a2 = acc[:, DQK // 2:]
    rot = jnp.concatenate([-a2, a1], axis=1)
    roped = acc * c + rot * s
    o_ref[...] = jnp.where(j < N_ROPE_TILES, roped, acc)


def _proj_rope(x, w_qkvw, cos, sin):
    TS = 2048
    grid = (S // TS, pl.cdiv(OUT_DIM, DQK))
    return pl.pallas_call(
        _proj_rope_body,
        grid=grid,
        in_specs=[
            pl.BlockSpec((TS, HID), lambda i, j: (i, 0)),
            pl.BlockSpec((DQK, HID), lambda i, j: (j, 0)),
            pl.BlockSpec((TS, DQK), lambda i, j: (i, 0)),
            pl.BlockSpec((TS, DQK), lambda i, j: (i, 0)),
        ],
        out_specs=pl.BlockSpec((TS, DQK), lambda i, j: (i, j)),
        out_shape=jax.ShapeDtypeStruct((S, OUT_DIM), jnp.float32),
    )(x, w_qkvw, cos, sin)


def _pool_body(k_ref, v_ref, kc_ref, vc_ref):
    kc_ref[0] = k_ref[0].reshape(T, BLOCK, DQK).mean(axis=1)
    vc_ref[0] = v_ref[0].reshape(T, BLOCK, DV).mean(axis=1)


def _pool(k, v):
    return pl.pallas_call(
        _pool_body,
        grid=(NKV,),
        in_specs=[
            pl.BlockSpec((1, S, DQK), lambda h: (h, 0, 0)),
            pl.BlockSpec((1, S, DV), lambda h: (h, 0, 0)),
        ],
        out_specs=[
            pl.BlockSpec((1, T, DQK), lambda h: (h, 0, 0)),
            pl.BlockSpec((1, T, DV), lambda h: (h, 0, 0)),
        ],
        out_shape=[
            jax.ShapeDtypeStruct((NKV, T, DQK), jnp.float32),
            jax.ShapeDtypeStruct((NKV, T, DV), jnp.float32),
        ],
    )(k, v)


def _attn_body(q_ref, k_ref, v_ref, kc_ref, vc_ref, w_ref, o_ref,
               ocmp_ref, sel_ref, qh_ref,
               msel_ref, lsel_ref, asel_ref,
               mwin_ref, lwin_ref, awin_ref):
    i = pl.program_id(1)
    j = pl.program_id(2)
    q0 = i * QB

    @pl.when(j == 0)
    def _first():
        qg = q_ref[...].reshape(RG, DQK)
        kc = kc_ref[0]
        vc = vc_ref[0]
        s_cmp = jax.lax.dot_general(
            qg, kc, (((1,), (1,)), ((), ())),
            preferred_element_type=jnp.float32) * _SCALE
        iic = q0 + jax.lax.broadcasted_iota(jnp.int32, (RG, T), 0) // G
        jbc = jax.lax.broadcasted_iota(jnp.int32, (RG, T), 1)
        s_cmp = jnp.where(jbc * BLOCK <= iic, s_cmp, -1e9)
        m = jnp.max(s_cmp, axis=1, keepdims=True)
        p = jnp.exp(s_cmp - m)
        p = p / jnp.sum(p, axis=1, keepdims=True)
        ocmp_ref[...] = jnp.dot(p, vc, preferred_element_type=jnp.float32)

        imp = p.reshape(QB, G, T).sum(axis=1)
        iota_f = jax.lax.broadcasted_iota(
            jnp.int32, (QB, T), 1).astype(jnp.float32)
        selk = jnp.zeros((QB, T), jnp.float32)
        work = imp
        for _ in range(TOPN):
            mx = jnp.max(work, axis=1, keepdims=True)
            cand = jnp.where(work == mx, iota_f, jnp.float32(T))
            js = jnp.min(cand, axis=1, keepdims=True)
            hit = iota_f == js
            selk = jnp.where(hit, 1.0, selk)
            work = jnp.where(hit, -jnp.inf, work)
        selg = jnp.broadcast_to(selk[:, None, :], (QB, G, T)).reshape(RG, T)
        sel_ref[...] = selg

        qh_ref[...] = (qg * _SCALE).astype(jnp.bfloat16)
        msel_ref[...] = jnp.full((RG, 1), -1e9, jnp.float32)
        lsel_ref[...] = jnp.zeros((RG, 1), jnp.float32)
        asel_ref[...] = jnp.zeros((RG, DV), jnp.float32)
        mwin_ref[...] = jnp.full((RG, 1), -1e9, jnp.float32)
        lwin_ref[...] = jnp.zeros((RG, 1), jnp.float32)
        awin_ref[...] = jnp.zeros((RG, DV), jnp.float32)

    @pl.when(j <= i)
    def _chunk():
        qh = qh_ref[...]
        kh = k_ref[0].astype(jnp.bfloat16)
        vh = v_ref[0].astype(jnp.bfloat16)
        ones8 = jnp.ones((KB, 8), jnp.bfloat16)
        s = jax.lax.dot_general(
            qh, kh, (((1,), (1,)), ((), ())),
            preferred_element_type=jnp.float32).astype(jnp.bfloat16)

        def _mask_diag(sv):
            cc = j * KB + jax.lax.broadcasted_iota(jnp.int32, (RG, KB), 1)
            rr = q0 + jax.lax.broadcasted_iota(jnp.int32, (RG, KB), 0) // G
            return jnp.where(cc <= rr, sv, jnp.bfloat16(-1e9))

        s = jax.lax.cond(j == i, _mask_diag, lambda sv: sv, s)

        mc16 = jnp.max(s, axis=1, keepdims=True)
        mc = mc16.astype(jnp.float32)
        e16 = jnp.exp(s - mc16)

        tt = jax.lax.broadcasted_iota(jnp.int32, (T, KB), 0)
        kb = j * BPC + jax.lax.broadcasted_iota(jnp.int32, (T, KB), 1) // BLOCK
        onehot = (tt == kb).astype(jnp.float32)
        selm = jax.lax.dot_general(
            sel_ref[...], onehot, (((1,), (0,)), ((), ())),
            preferred_element_type=jnp.float32).astype(jnp.bfloat16)

        def _update(p16, m_ref, l_ref, a_ref):
            m_old = m_ref[...]
            m_new = jnp.maximum(m_old, mc)
            alpha = jnp.exp(m_old - m_new)
            beta = jnp.exp(mc - m_new)
            rsum = jax.lax.dot_general(
                p16, ones8, (((1,), (0,)), ((), ())),
                preferred_element_type=jnp.float32)[:, 0:1]
            pv = jax.lax.dot_general(
                p16, vh, (((1,), (0,)), ((), ())),
                preferred_element_type=jnp.float32)
            l_ref[...] = l_ref[...] * alpha + rsum * beta
            a_ref[...] = a_ref[...] * alpha + pv * beta
            m_ref[...] = m_new

        _update(e16 * selm, msel_ref, lsel_ref, asel_ref)

        @pl.when(j + WCH >= i)
        def _win():
            def _mask_band(ev):
                rr = q0 + jax.lax.broadcasted_iota(jnp.int32, (RG, KB), 0) // G
                cc = j * KB + jax.lax.broadcasted_iota(jnp.int32, (RG, KB), 1)
                wm = jnp.where(rr - cc < WIN, 1.0, 0.0).astype(jnp.bfloat16)
                return ev * wm

            pw = jax.lax.cond(j + WCH == i, _mask_band, lambda ev: ev, e16)
            _update(pw, mwin_ref, lwin_ref, awin_ref)

        @pl.when(j == i)
        def _fin():
            g = jax.nn.sigmoid(w_ref[...].reshape(RG, 3))
            o_sel = asel_ref[...] * (g[:, 1:2] / lsel_ref[...])
            o_win = awin_ref[...] * (g[:, 2:3] / lwin_ref[...])
            out = g[:, 0:1] * ocmp_ref[...] + o_sel + o_win
            o_ref[...] = out.reshape(QB, 1, G, DV)


def _attn(q, k, v, kc, vc, w):
    grid = (NKV, NQ, NKJ)
    return pl.pallas_call(
        _attn_body,
        grid=grid,
        in_specs=[
            pl.BlockSpec((QB, 1, G, DQK), lambda h, i, j: (i, h, 0, 0)),
            pl.BlockSpec((1, KB, DQK),
                         lambda h, i, j: (h, jnp.minimum(j, i), 0)),
            pl.BlockSpec((1, KB, DV),
                         lambda h, i, j: (h, jnp.minimum(j, i), 0)),
            pl.BlockSpec((1, T, DQK), lambda h, i, j: (h, 0, 0)),
            pl.BlockSpec((1, T, DV), lambda h, i, j: (h, 0, 0)),
            pl.BlockSpec((QB, 1, G, 3), lambda h, i, j: (i, h, 0, 0)),
        ],
        out_specs=pl.BlockSpec((QB, 1, G, DV), lambda h, i, j: (i, h, 0, 0)),
        out_shape=jax.ShapeDtypeStruct((S, NKV, G, DV), jnp.float32),
        scratch_shapes=[
            pltpu.VMEM((RG, DV), jnp.float32),
            pltpu.VMEM((RG, T), jnp.float32),
            pltpu.VMEM((RG, DQK), jnp.bfloat16),
            pltpu.VMEM((RG, 1), jnp.float32),
            pltpu.VMEM((RG, 1), jnp.float32),
            pltpu.VMEM((RG, DV), jnp.float32),
            pltpu.VMEM((RG, 1), jnp.float32),
            pltpu.VMEM((RG, 1), jnp.float32),
            pltpu.VMEM((RG, DV), jnp.float32),
        ],
    )(q, k, v, kc, vc, w)


def _out_proj_body(x_ref, w_ref, o_ref):
    o_ref[...] = jax.lax.dot_general(
        x_ref[...], w_ref[...],
        (((1,), (1,)), ((), ())),
        preferred_element_type=jnp.float32)


def _out_proj(x, w_o):
    TS = 2048
    TN = 512
    grid = (S // TS, HID // TN)
    return pl.pallas_call(
        _out_proj_body,
        grid=grid,
        in_specs=[
            pl.BlockSpec((TS, NH * DV), lambda i, j: (i, 0)),
            pl.BlockSpec((TN, NH * DV), lambda i, j: (j, 0)),
        ],
        out_specs=pl.BlockSpec((TS, TN), lambda i, j: (i, j)),
        out_shape=jax.ShapeDtypeStruct((S, HID), jnp.float32),
    )(x, w_o)


def kernel(hidden_states, cos, sin, cu_seqlens, W_qkvw, W_o):
    s, b, d = hidden_states.shape
    x = hidden_states.reshape(s, d)
    qkvw = _proj_rope(x, W_qkvw, cos, sin)
    o0 = NH * DQK
    o1 = o0 + NKV * DQK
    o2 = o1 + NKV * DV
    q = qkvw[:, :o0].reshape(S, NKV, G, DQK)
    k = qkvw[:, o0:o1].reshape(S, NKV, DQK).transpose(1, 0, 2)
    v = qkvw[:, o1:o2].reshape(S, NKV, DV).transpose(1, 0, 2)
    w = qkvw[:, o2:].reshape(S, NKV, G, 3)
    kc, vc = _pool(k, v)
    o = _attn(q, k, v, kc, vc, w)
    out = _out_proj(o.reshape(S, NH * DV), W_o)
    return out.reshape(s, b, HID)
